# baseline (device time: 216901 ns/iter reference)
import jax
import jax.numpy as jnp
from jax import lax
from jax.experimental import pallas as pl
from jax.experimental.pallas import tpu as pltpu

N_DEV = 16
B_PER = 2
SQ = 128
SKV = 128
HQ = 64
HQ_PER = 4
DH = 64
D_MODEL = 512
HD_PER = HQ_PER * DH
TOK = B_PER * SQ
SCALE = 0.125


def kernel(x, Wq, K_ext, V_ext, Wo):
    my = lax.axis_index("i")

    w_cat = jnp.concatenate([Wq, Wo.T], axis=1)

    k_loc = lax.dynamic_slice(K_ext, (my * B_PER, 0, 0, 0), (B_PER, SKV, HQ, DH))
    v_loc = lax.dynamic_slice(V_ext, (my * B_PER, 0, 0, 0), (B_PER, SKV, HQ, DH))
    perm = (my - jnp.arange(N_DEV)) % N_DEV
    k_arr = k_loc.reshape(B_PER, SKV, N_DEV, HQ_PER, DH).transpose(2, 0, 1, 3, 4)[perm]
    v_arr = v_loc.reshape(B_PER, SKV, N_DEV, HQ_PER, DH).transpose(2, 0, 1, 3, 4)[perm]

    def body(x_ref, w_ref, k_ref, v_ref, out_ref, comm_ref, send_sems, recv_sems):
        me = lax.axis_index("i")
        left = (me + N_DEV - 1) % N_DEV
        right = (me + 1) % N_DEV

        barrier_sem = pltpu.get_barrier_semaphore()
        for nbr in (left, right):
            pl.semaphore_signal(
                barrier_sem, inc=1,
                device_id=(nbr,), device_id_type=pl.DeviceIdType.MESH,
            )
        pl.semaphore_wait(barrier_sem, 2)

        x2 = x_ref[...].reshape(TOK, D_MODEL)

        def compute(t, w):
            wq = w[:, :HD_PER]
            woT = w[:, HD_PER:]
            q4 = jnp.dot(x2, wq, preferred_element_type=jnp.float32)
            q4 = q4.reshape(B_PER, SQ, HQ_PER, DH)
            k5 = k_ref[t]
            v5 = v_ref[t]
            rows = []
            for b in range(B_PER):
                ctx_h = []
                for hh in range(HQ_PER):
                    qbh = q4[b, :, hh, :]
                    kbh = k5[b, :, hh, :]
                    s = lax.dot_general(
                        qbh, kbh, (((1,), (1,)), ((), ())),
                        preferred_element_type=jnp.float32,
                    ) * SCALE
                    s = s - jnp.max(s, axis=-1, keepdims=True)
                    e = jnp.exp(s)
                    p = e / jnp.sum(e, axis=-1, keepdims=True)
                    ctx_h.append(
                        jnp.dot(p, v5[b, :, hh, :], preferred_element_type=jnp.float32)
                    )
                rows.append(jnp.concatenate(ctx_h, axis=1))
            ctx2 = jnp.concatenate(rows, axis=0)
            return lax.dot_general(
                ctx2, woT, (((1,), (1,)), ((), ())),
                preferred_element_type=jnp.float32,
            )

        rdmas = []
        acc = None
        for h in range(1, N_DEV):
            src = w_ref if h == 1 else comm_ref.at[h - 2]
            rdma = pltpu.make_async_remote_copy(
                src_ref=src,
                dst_ref=comm_ref.at[h - 1],
                send_sem=send_sems.at[h - 1],
                recv_sem=recv_sems.at[h - 1],
                device_id=(right,),
                device_id_type=pl.DeviceIdType.MESH,
            )
            rdma.start()
            rdmas.append(rdma)
            w_prev = w_ref[...] if h == 1 else comm_ref[h - 2]
            part = compute(h - 1, w_prev)
            acc = part if acc is None else acc + part
            rdma.wait_recv()
        acc = acc + compute(N_DEV - 1, comm_ref[N_DEV - 2])
        for rdma in rdmas:
            rdma.wait_send()

        out_ref[...] = acc.reshape(B_PER, SQ, D_MODEL)

    return pl.pallas_call(
        body,
        out_shape=jax.ShapeDtypeStruct((B_PER, SQ, D_MODEL), jnp.float32),
        in_specs=[
            pl.BlockSpec(memory_space=pltpu.VMEM),
            pl.BlockSpec(memory_space=pltpu.VMEM),
            pl.BlockSpec(memory_space=pltpu.VMEM),
            pl.BlockSpec(memory_space=pltpu.VMEM),
        ],
        out_specs=pl.BlockSpec(memory_space=pltpu.VMEM),
        scratch_shapes=[
            pltpu.VMEM((N_DEV - 1, D_MODEL, 2 * HD_PER), jnp.float32),
            pltpu.SemaphoreType.DMA((N_DEV - 1,)),
            pltpu.SemaphoreType.DMA((N_DEV - 1,)),
        ],
        compiler_params=pltpu.CompilerParams(collective_id=0),
    )(x, w_cat, k_arr, v_arr)


# device time: 82839 ns/iter; 2.6183x vs baseline; 2.6183x over previous
import jax
import jax.numpy as jnp
from jax import lax
from jax.experimental import pallas as pl
from jax.experimental.pallas import tpu as pltpu

N_DEV = 16
N_CW = 8
N_CCW = 7
B_PER = 2
SQ = 128
SKV = 128
HQ = 64
HQ_PER = 4
DH = 64
D_MODEL = 512
HD_PER = HQ_PER * DH
TOK = B_PER * SQ
SCALE = 0.125

_OFFS = [0]
for _r in range(1, 8):
    _OFFS += [-_r, _r]
_OFFS.append(-8)


def kernel(x, Wq, K_ext, V_ext, Wo):
    my = lax.axis_index("i")

    w_cat = jnp.concatenate([Wq, Wo.T], axis=1).astype(jnp.bfloat16)

    k_loc = lax.dynamic_slice(K_ext, (my * B_PER, 0, 0, 0), (B_PER, SKV, HQ, DH))
    v_loc = lax.dynamic_slice(V_ext, (my * B_PER, 0, 0, 0), (B_PER, SKV, HQ, DH))
    perm = (my + jnp.array(_OFFS)) % N_DEV
    k_arr = k_loc.reshape(B_PER, SKV, N_DEV, HQ_PER, DH).transpose(2, 0, 1, 3, 4)[perm]
    v_arr = v_loc.reshape(B_PER, SKV, N_DEV, HQ_PER, DH).transpose(2, 0, 1, 3, 4)[perm]

    def body(x_ref, w_ref, k_ref, v_ref, out_ref,
             cw_ref, ccw_ref, cw_send, cw_recv, ccw_send, ccw_recv):
        me = lax.axis_index("i")
        left = (me + N_DEV - 1) % N_DEV
        right = (me + 1) % N_DEV

        barrier_sem = pltpu.get_barrier_semaphore()
        for nbr in (left, right):
            pl.semaphore_signal(
                barrier_sem, inc=1,
                device_id=(nbr,), device_id_type=pl.DeviceIdType.MESH,
            )
        pl.semaphore_wait(barrier_sem, 2)

        x2 = x_ref[...].reshape(TOK, D_MODEL)

        def compute(t, w):
            wq = w[:, :HD_PER]
            woT = w[:, HD_PER:]
            q4 = jnp.dot(x2, wq, preferred_element_type=jnp.float32)
            q4 = q4.reshape(B_PER, SQ, HQ_PER, DH)
            k5 = k_ref[t]
            v5 = v_ref[t]
            rows = []
            for b in range(B_PER):
                ctx_h = []
                for hh in range(HQ_PER):
                    qbh = q4[b, :, hh, :]
                    kbh = k5[b, :, hh, :]
                    s = lax.dot_general(
                        qbh, kbh, (((1,), (1,)), ((), ())),
                        preferred_element_type=jnp.float32,
                    ) * SCALE
                    s = s - jnp.max(s, axis=-1, keepdims=True)
                    e = jnp.exp(s)
                    p = e / jnp.sum(e, axis=-1, keepdims=True)
                    ctx_h.append(
                        jnp.dot(p, v5[b, :, hh, :], preferred_element_type=jnp.float32)
                    )
                rows.append(jnp.concatenate(ctx_h, axis=1))
            ctx2 = jnp.concatenate(rows, axis=0)
            return lax.dot_general(
                ctx2, woT, (((1,), (1,)), ((), ())),
                preferred_element_type=jnp.float32,
            )

        def mk(src, dst_ref, slot, send, recv, dst_dev):
            return pltpu.make_async_remote_copy(
                src_ref=src,
                dst_ref=dst_ref.at[slot],
                send_sem=send.at[slot],
                recv_sem=recv.at[slot],
                device_id=(dst_dev,),
                device_id_type=pl.DeviceIdType.MESH,
            )

        cw = [mk(w_ref, cw_ref, 0, cw_send, cw_recv, right)]
        ccw = [mk(w_ref, ccw_ref, 0, ccw_send, ccw_recv, left)]
        cw[0].start()
        ccw[0].start()

        acc = compute(0, w_ref[...])

        for r in range(1, N_CW + 1):
            cw[r - 1].wait_recv()
            if r < N_CW:
                nxt = mk(cw_ref.at[r - 1], cw_ref, r, cw_send, cw_recv, right)
                nxt.start()
                cw.append(nxt)
            acc = acc + compute(2 * r - 1 if r < N_CW else 15, cw_ref[r - 1])
            if r <= N_CCW:
                ccw[r - 1].wait_recv()
                if r < N_CCW:
                    nxt = mk(ccw_ref.at[r - 1], ccw_ref, r, ccw_send, ccw_recv, left)
                    nxt.start()
                    ccw.append(nxt)
                acc = acc + compute(2 * r, ccw_ref[r - 1])

        for rdma in cw + ccw:
            rdma.wait_send()

        out_ref[...] = acc.reshape(B_PER, SQ, D_MODEL)

    return pl.pallas_call(
        body,
        out_shape=jax.ShapeDtypeStruct((B_PER, SQ, D_MODEL), jnp.float32),
        in_specs=[
            pl.BlockSpec(memory_space=pltpu.VMEM),
            pl.BlockSpec(memory_space=pltpu.VMEM),
            pl.BlockSpec(memory_space=pltpu.VMEM),
            pl.BlockSpec(memory_space=pltpu.VMEM),
        ],
        out_specs=pl.BlockSpec(memory_space=pltpu.VMEM),
        scratch_shapes=[
            pltpu.VMEM((N_CW, D_MODEL, 2 * HD_PER), jnp.bfloat16),
            pltpu.VMEM((N_CCW, D_MODEL, 2 * HD_PER), jnp.bfloat16),
            pltpu.SemaphoreType.DMA((N_CW,)),
            pltpu.SemaphoreType.DMA((N_CW,)),
            pltpu.SemaphoreType.DMA((N_CCW,)),
            pltpu.SemaphoreType.DMA((N_CCW,)),
        ],
        compiler_params=pltpu.CompilerParams(collective_id=0),
    )(x, w_cat, k_arr, v_arr)


# device time: 80148 ns/iter; 2.7063x vs baseline; 1.0336x over previous
import jax
import jax.numpy as jnp
from jax import lax
from jax.experimental import pallas as pl
from jax.experimental.pallas import tpu as pltpu

N_DEV = 16
N_CW = 8
N_CCW = 7
B_PER = 2
SQ = 128
SKV = 128
HQ = 64
HQ_PER = 4
DH = 64
D_MODEL = 512
HD_PER = HQ_PER * DH
TOK = B_PER * SQ
SCALE = 0.125

_OFFS = [0]
for _r in range(1, 8):
    _OFFS += [-_r, _r]
_OFFS.append(-8)


def kernel(x, Wq, K_ext, V_ext, Wo):
    my = lax.axis_index("i")

    w_cat = jnp.concatenate([Wq * SCALE, Wo.T], axis=1).astype(jnp.bfloat16)

    k_loc = lax.dynamic_slice(K_ext, (my * B_PER, 0, 0, 0), (B_PER, SKV, HQ, DH))
    v_loc = lax.dynamic_slice(V_ext, (my * B_PER, 0, 0, 0), (B_PER, SKV, HQ, DH))
    perm = (my + jnp.array(_OFFS)) % N_DEV
    k_arr = k_loc.reshape(B_PER, SKV, N_DEV, HQ_PER, DH).transpose(2, 0, 1, 3, 4)[perm]
    v_arr = v_loc.reshape(B_PER, SKV, N_DEV, HQ_PER, DH).transpose(2, 0, 1, 3, 4)[perm]
    k_arr = k_arr.astype(jnp.bfloat16)
    v_arr = v_arr.astype(jnp.bfloat16)

    def body(x_ref, w_ref, k_ref, v_ref, out_ref,
             cw_ref, ccw_ref, cw_send, cw_recv, ccw_send, ccw_recv):
        me = lax.axis_index("i")
        left = (me + N_DEV - 1) % N_DEV
        right = (me + 1) % N_DEV

        barrier_sem = pltpu.get_barrier_semaphore()
        for nbr in (left, right):
            pl.semaphore_signal(
                barrier_sem, inc=1,
                device_id=(nbr,), device_id_type=pl.DeviceIdType.MESH,
            )
        pl.semaphore_wait(barrier_sem, 2)

        x2 = x_ref[...].reshape(TOK, D_MODEL).astype(jnp.bfloat16)

        def compute(t, w):
            wq = w[:, :HD_PER]
            woT = w[:, HD_PER:]
            q4 = jnp.dot(x2, wq, preferred_element_type=jnp.float32)
            q4 = q4.reshape(B_PER, SQ, HQ_PER, DH).astype(jnp.bfloat16)
            k5 = k_ref[t]
            v5 = v_ref[t]
            rows = []
            for b in range(B_PER):
                ctx_h = []
                for hh in range(HQ_PER):
                    qbh = q4[b, :, hh, :]
                    kbh = k5[b, :, hh, :]
                    s = lax.dot_general(
                        qbh, kbh, (((1,), (1,)), ((), ())),
                        preferred_element_type=jnp.float32,
                    )
                    e = jnp.exp(s)
                    r = 1.0 / jnp.sum(e, axis=-1, keepdims=True)
                    ev = jnp.dot(
                        e.astype(jnp.bfloat16), v5[b, :, hh, :],
                        preferred_element_type=jnp.float32,
                    )
                    ctx_h.append(ev * r)
                rows.append(jnp.concatenate(ctx_h, axis=1))
            ctx2 = jnp.concatenate(rows, axis=0).astype(jnp.bfloat16)
            return lax.dot_general(
                ctx2, woT, (((1,), (1,)), ((), ())),
                preferred_element_type=jnp.float32,
            )

        def mk(src, dst_ref, slot, send, recv, dst_dev):
            return pltpu.make_async_remote_copy(
                src_ref=src,
                dst_ref=dst_ref.at[slot],
                send_sem=send.at[slot],
                recv_sem=recv.at[slot],
                device_id=(dst_dev,),
                device_id_type=pl.DeviceIdType.MESH,
            )

        cw = [mk(w_ref, cw_ref, 0, cw_send, cw_recv, right)]
        ccw = [mk(w_ref, ccw_ref, 0, ccw_send, ccw_recv, left)]
        cw[0].start()
        ccw[0].start()

        acc = compute(0, w_ref[...])

        for r in range(1, N_CW + 1):
            cw[r - 1].wait_recv()
            if r < N_CW:
                nxt = mk(cw_ref.at[r - 1], cw_ref, r, cw_send, cw_recv, right)
                nxt.start()
                cw.append(nxt)
            acc = acc + compute(2 * r - 1 if r < N_CW else 15, cw_ref[r - 1])
            if r <= N_CCW:
                ccw[r - 1].wait_recv()
                if r < N_CCW:
                    nxt = mk(ccw_ref.at[r - 1], ccw_ref, r, ccw_send, ccw_recv, left)
                    nxt.start()
                    ccw.append(nxt)
                acc = acc + compute(2 * r, ccw_ref[r - 1])

        for rdma in cw + ccw:
            rdma.wait_send()

        out_ref[...] = acc.reshape(B_PER, SQ, D_MODEL)

    return pl.pallas_call(
        body,
        out_shape=jax.ShapeDtypeStruct((B_PER, SQ, D_MODEL), jnp.float32),
        in_specs=[
            pl.BlockSpec(memory_space=pltpu.VMEM),
            pl.BlockSpec(memory_space=pltpu.VMEM),
            pl.BlockSpec(memory_space=pltpu.VMEM),
            pl.BlockSpec(memory_space=pltpu.VMEM),
        ],
        out_specs=pl.BlockSpec(memory_space=pltpu.VMEM),
        scratch_shapes=[
            pltpu.VMEM((N_CW, D_MODEL, 2 * HD_PER), jnp.bfloat16),
            pltpu.VMEM((N_CCW, D_MODEL, 2 * HD_PER), jnp.bfloat16),
            pltpu.SemaphoreType.DMA((N_CW,)),
            pltpu.SemaphoreType.DMA((N_CW,)),
            pltpu.SemaphoreType.DMA((N_CCW,)),
            pltpu.SemaphoreType.DMA((N_CCW,)),
        ],
        compiler_params=pltpu.CompilerParams(collective_id=0),
    )(x, w_cat, k_arr, v_arr)


# device time: 78535 ns/iter; 2.7618x vs baseline; 1.0205x over previous
import jax
import jax.numpy as jnp
from jax import lax
from jax.experimental import pallas as pl
from jax.experimental.pallas import tpu as pltpu

N_DEV = 16
N_CW = 8
N_CCW = 7
B_PER = 2
SQ = 128
SKV = 128
HQ = 64
HQ_PER = 4
DH = 64
D_MODEL = 512
HD_PER = HQ_PER * DH
TOK = B_PER * SQ
SCALE = 0.125

_OFFS = [0]
for _r in range(1, 8):
    _OFFS += [-_r, _r]
_OFFS.append(-8)


def kernel(x, Wq, K_ext, V_ext, Wo):
    my = lax.axis_index("i")

    w_cat = jnp.concatenate([Wq * SCALE, Wo.T], axis=1).astype(jnp.bfloat16)

    k_loc = lax.dynamic_slice(K_ext, (my * B_PER, 0, 0, 0), (B_PER, SKV, HQ, DH))
    v_loc = lax.dynamic_slice(V_ext, (my * B_PER, 0, 0, 0), (B_PER, SKV, HQ, DH))
    perm = (my + jnp.array(_OFFS)) % N_DEV
    k_arr = k_loc.reshape(B_PER, SKV, N_DEV, HQ_PER, DH).transpose(2, 0, 3, 1, 4)[perm]
    v_arr = v_loc.reshape(B_PER, SKV, N_DEV, HQ_PER, DH).transpose(2, 0, 3, 1, 4)[perm]
    k_arr = k_arr.astype(jnp.bfloat16)
    v_arr = v_arr.astype(jnp.bfloat16)

    def body(x_ref, w_ref, k_ref, v_ref, out_ref,
             cw_ref, ccw_ref, cw_send, cw_recv, ccw_send, ccw_recv):
        me = lax.axis_index("i")
        left = (me + N_DEV - 1) % N_DEV
        right = (me + 1) % N_DEV

        barrier_sem = pltpu.get_barrier_semaphore()
        for nbr in (left, right):
            pl.semaphore_signal(
                barrier_sem, inc=1,
                device_id=(nbr,), device_id_type=pl.DeviceIdType.MESH,
            )
        pl.semaphore_wait(barrier_sem, 2)

        x2 = x_ref[...].reshape(TOK, D_MODEL).astype(jnp.bfloat16)

        def compute(t, w):
            wq = w[:, :HD_PER]
            woT = w[:, HD_PER:]
            q4 = jnp.dot(x2, wq, preferred_element_type=jnp.float32)
            q4 = q4.reshape(B_PER, SQ, HQ_PER, DH).astype(jnp.bfloat16)
            k5 = k_ref[t]
            v5 = v_ref[t]
            rows = []
            for b in range(B_PER):
                ctx_h = []
                for hh in range(HQ_PER):
                    qbh = q4[b, :, hh, :]
                    kbh = k5[b, hh]
                    s = lax.dot_general(
                        qbh, kbh, (((1,), (1,)), ((), ())),
                        preferred_element_type=jnp.float32,
                    )
                    e = jnp.exp(s)
                    r = 1.0 / jnp.sum(e, axis=-1, keepdims=True)
                    ev = jnp.dot(
                        e.astype(jnp.bfloat16), v5[b, hh],
                        preferred_element_type=jnp.float32,
                    )
                    ctx_h.append(ev * r)
                rows.append(jnp.concatenate(ctx_h, axis=1))
            ctx2 = jnp.concatenate(rows, axis=0).astype(jnp.bfloat16)
            return lax.dot_general(
                ctx2, woT, (((1,), (1,)), ((), ())),
                preferred_element_type=jnp.float32,
            )

        def mk(src, dst_ref, slot, send, recv, dst_dev):
            return pltpu.make_async_remote_copy(
                src_ref=src,
                dst_ref=dst_ref.at[slot],
                send_sem=send.at[slot],
                recv_sem=recv.at[slot],
                device_id=(dst_dev,),
                device_id_type=pl.DeviceIdType.MESH,
            )

        cw = [mk(w_ref, cw_ref, 0, cw_send, cw_recv, right)]
        ccw = [mk(w_ref, ccw_ref, 0, ccw_send, ccw_recv, left)]
        cw[0].start()
        ccw[0].start()

        acc = compute(0, w_ref[...])

        for r in range(1, N_CW + 1):
            cw[r - 1].wait_recv()
            if r < N_CW:
                nxt = mk(cw_ref.at[r - 1], cw_ref, r, cw_send, cw_recv, right)
                nxt.start()
                cw.append(nxt)
            acc = acc + compute(2 * r - 1 if r < N_CW else 15, cw_ref[r - 1])
            if r <= N_CCW:
                ccw[r - 1].wait_recv()
                if r < N_CCW:
                    nxt = mk(ccw_ref.at[r - 1], ccw_ref, r, ccw_send, ccw_recv, left)
                    nxt.start()
                    ccw.append(nxt)
                acc = acc + compute(2 * r, ccw_ref[r - 1])

        for rdma in cw + ccw:
            rdma.wait_send()

        out_ref[...] = acc.reshape(B_PER, SQ, D_MODEL)

    return pl.pallas_call(
        body,
        out_shape=jax.ShapeDtypeStruct((B_PER, SQ, D_MODEL), jnp.float32),
        in_specs=[
            pl.BlockSpec(memory_space=pltpu.VMEM),
            pl.BlockSpec(memory_space=pltpu.VMEM),
            pl.BlockSpec(memory_space=pltpu.VMEM),
            pl.BlockSpec(memory_space=pltpu.VMEM),
        ],
        out_specs=pl.BlockSpec(memory_space=pltpu.VMEM),
        scratch_shapes=[
            pltpu.VMEM((N_CW, D_MODEL, 2 * HD_PER), jnp.bfloat16),
            pltpu.VMEM((N_CCW, D_MODEL, 2 * HD_PER), jnp.bfloat16),
            pltpu.SemaphoreType.DMA((N_CW,)),
            pltpu.SemaphoreType.DMA((N_CW,)),
            pltpu.SemaphoreType.DMA((N_CCW,)),
            pltpu.SemaphoreType.DMA((N_CCW,)),
        ],
        compiler_params=pltpu.CompilerParams(collective_id=0),
    )(x, w_cat, k_arr, v_arr)


# device time: 72172 ns/iter; 3.0053x vs baseline; 1.0882x over previous
import jax
import jax.numpy as jnp
from jax import lax
from jax.experimental import pallas as pl
from jax.experimental.pallas import tpu as pltpu

N_DEV = 16
N_CW = 8
N_CCW = 7
B_PER = 2
SQ = 128
SKV = 128
HQ = 64
HQ_PER = 4
DH = 64
D_MODEL = 512
HD_PER = HQ_PER * DH
TOK = B_PER * SQ
SCALE = 0.125

_OFFS = [0]
for _r in range(1, 8):
    _OFFS += [-_r, _r]
_OFFS.append(-8)


def kernel(x, Wq, K_ext, V_ext, Wo):
    my = lax.axis_index("i")

    w_cat = jnp.concatenate([Wq * SCALE, Wo.T], axis=1).astype(jnp.bfloat16)

    k_loc = lax.dynamic_slice(K_ext, (my * B_PER, 0, 0, 0), (B_PER, SKV, HQ, DH))
    v_loc = lax.dynamic_slice(V_ext, (my * B_PER, 0, 0, 0), (B_PER, SKV, HQ, DH))
    perm = (my + jnp.array(_OFFS)) % N_DEV
    k_arr = k_loc.reshape(B_PER, SKV, N_DEV, HQ_PER, DH).transpose(2, 0, 3, 1, 4)[perm]
    v_arr = v_loc.reshape(B_PER, SKV, N_DEV, HQ_PER, DH).transpose(2, 0, 3, 1, 4)[perm]
    k_arr = k_arr.astype(jnp.bfloat16)
    v_arr = v_arr.astype(jnp.bfloat16)

    def body(x_ref, w_ref, k_ref, v_ref, out_ref,
             cw_ref, ccw_ref, cw_send, cw_recv, ccw_send, ccw_recv):
        me = lax.axis_index("i")
        left = (me + N_DEV - 1) % N_DEV
        right = (me + 1) % N_DEV

        barrier_sem = pltpu.get_barrier_semaphore()
        for nbr in (left, right):
            pl.semaphore_signal(
                barrier_sem, inc=1,
                device_id=(nbr,), device_id_type=pl.DeviceIdType.MESH,
            )
        pl.semaphore_wait(barrier_sem, 2)

        x2 = x_ref[...].reshape(TOK, D_MODEL).astype(jnp.bfloat16)

        def compute(t, w):
            wq = w[:, :HD_PER]
            woT = w[:, HD_PER:]
            q4 = jnp.dot(x2, wq, preferred_element_type=jnp.float32)
            q4 = q4.reshape(B_PER, SQ, HQ_PER, DH).astype(jnp.bfloat16)
            k5 = k_ref[t]
            v5 = v_ref[t]
            rows = []
            for b in range(B_PER):
                ctx_h = []
                for hh in range(HQ_PER):
                    qbh = q4[b, :, hh, :]
                    kbh = k5[b, hh]
                    s = lax.dot_general(
                        qbh, kbh, (((1,), (1,)), ((), ())),
                        preferred_element_type=jnp.float32,
                    )
                    e = jnp.exp(s)
                    r = 1.0 / jnp.sum(e, axis=-1, keepdims=True)
                    ev = jnp.dot(
                        e.astype(jnp.bfloat16), v5[b, hh],
                        preferred_element_type=jnp.float32,
                    )
                    ctx_h.append(ev * r)
                rows.append(jnp.concatenate(ctx_h, axis=1))
            ctx2 = jnp.concatenate(rows, axis=0).astype(jnp.bfloat16)
            return lax.dot_general(
                ctx2, woT, (((1,), (1,)), ((), ())),
                preferred_element_type=jnp.float32,
            )

        HALF = D_MODEL // 2

        def mk(src_ref, src_slot, dst_ref, slot, half, send, recv, dst_dev):
            rows = pl.ds(half * HALF, HALF)
            src = (src_ref.at[rows] if src_slot is None
                   else src_ref.at[src_slot, rows])
            return pltpu.make_async_remote_copy(
                src_ref=src,
                dst_ref=dst_ref.at[slot, rows],
                send_sem=send.at[slot, half],
                recv_sem=recv.at[slot, half],
                device_id=(dst_dev,),
                device_id_type=pl.DeviceIdType.MESH,
            )

        cw, ccw = [], []
        for half in (0, 1):
            cw.append(mk(w_ref, None, cw_ref, 0, half, cw_send, cw_recv, right))
            ccw.append(mk(w_ref, None, ccw_ref, 0, half, ccw_send, ccw_recv, left))
            cw[-1].start()
            ccw[-1].start()

        acc = compute(0, w_ref[...])

        for r in range(1, N_CW + 1):
            has_ccw = r <= N_CCW
            for half in (0, 1):
                cw[2 * (r - 1) + half].wait_recv()
                if r < N_CW:
                    nxt = mk(cw_ref, r - 1, cw_ref, r, half, cw_send, cw_recv, right)
                    nxt.start()
                    cw.append(nxt)
                if has_ccw:
                    ccw[2 * (r - 1) + half].wait_recv()
                    if r < N_CCW:
                        nxt = mk(ccw_ref, r - 1, ccw_ref, r, half,
                                 ccw_send, ccw_recv, left)
                        nxt.start()
                        ccw.append(nxt)
            acc = acc + compute(2 * r - 1 if r < N_CW else 15, cw_ref[r - 1])
            if has_ccw:
                acc = acc + compute(2 * r, ccw_ref[r - 1])

        for rdma in cw + ccw:
            rdma.wait_send()

        out_ref[...] = acc.reshape(B_PER, SQ, D_MODEL)

    return pl.pallas_call(
        body,
        out_shape=jax.ShapeDtypeStruct((B_PER, SQ, D_MODEL), jnp.float32),
        in_specs=[
            pl.BlockSpec(memory_space=pltpu.VMEM),
            pl.BlockSpec(memory_space=pltpu.VMEM),
            pl.BlockSpec(memory_space=pltpu.VMEM),
            pl.BlockSpec(memory_space=pltpu.VMEM),
        ],
        out_specs=pl.BlockSpec(memory_space=pltpu.VMEM),
        scratch_shapes=[
            pltpu.VMEM((N_CW, D_MODEL, 2 * HD_PER), jnp.bfloat16),
            pltpu.VMEM((N_CCW, D_MODEL, 2 * HD_PER), jnp.bfloat16),
            pltpu.SemaphoreType.DMA((N_CW, 2)),
            pltpu.SemaphoreType.DMA((N_CW, 2)),
            pltpu.SemaphoreType.DMA((N_CCW, 2)),
            pltpu.SemaphoreType.DMA((N_CCW, 2)),
        ],
        compiler_params=pltpu.CompilerParams(collective_id=0),
    )(x, w_cat, k_arr, v_arr)


# device time: 67799 ns/iter; 3.1992x vs baseline; 1.0645x over previous
import jax
import jax.numpy as jnp
from jax import lax
from jax.experimental import pallas as pl
from jax.experimental.pallas import tpu as pltpu

N_DEV = 16
N_CW = 8
N_CCW = 7
B_PER = 2
SQ = 128
SKV = 128
HQ = 64
HQ_PER = 4
DH = 64
D_MODEL = 512
HD_PER = HQ_PER * DH
TOK = B_PER * SQ
SCALE = 0.125

_OFFS = [0]
for _r in range(1, 8):
    _OFFS += [-_r, _r]
_OFFS.append(-8)


def kernel(x, Wq, K_ext, V_ext, Wo):
    my = lax.axis_index("i")

    w_cat = jnp.concatenate([Wq * SCALE, Wo.T], axis=1).astype(jnp.bfloat16)

    k_loc = lax.dynamic_slice(K_ext, (my * B_PER, 0, 0, 0), (B_PER, SKV, HQ, DH))
    v_loc = lax.dynamic_slice(V_ext, (my * B_PER, 0, 0, 0), (B_PER, SKV, HQ, DH))
    perm = (my + jnp.array(_OFFS)) % N_DEV
    k_arr = k_loc.reshape(B_PER, SKV, N_DEV, HQ_PER, DH).transpose(2, 0, 3, 1, 4)[perm]
    v_arr = v_loc.reshape(B_PER, SKV, N_DEV, HQ_PER, DH).transpose(2, 0, 3, 1, 4)[perm]
    k_arr = k_arr.astype(jnp.bfloat16)
    v_arr = v_arr.astype(jnp.bfloat16)

    def body(x_ref, w_ref, k_ref, v_ref, out_ref,
             cw_ref, ccw_ref, cw_send, cw_recv, ccw_send, ccw_recv):
        me = lax.axis_index("i")
        left = (me + N_DEV - 1) % N_DEV
        right = (me + 1) % N_DEV

        barrier_sem = pltpu.get_barrier_semaphore()
        for nbr in (left, right):
            pl.semaphore_signal(
                barrier_sem, inc=1,
                device_id=(nbr,), device_id_type=pl.DeviceIdType.MESH,
            )
        pl.semaphore_wait(barrier_sem, 2)

        x2 = x_ref[...].reshape(TOK, D_MODEL).astype(jnp.bfloat16)

        def compute(t, w):
            wq = w[:, :HD_PER]
            woT = w[:, HD_PER:]
            q = jnp.dot(x2, wq, preferred_element_type=jnp.float32)
            q4t = (
                q.reshape(B_PER, SQ, HQ_PER, DH)
                .transpose(0, 2, 1, 3)
                .reshape(B_PER * HQ_PER, SQ, DH)
                .astype(jnp.bfloat16)
            )
            k5 = k_ref[t].reshape(B_PER * HQ_PER, SKV, DH)
            v5 = v_ref[t].reshape(B_PER * HQ_PER, SKV, DH)
            s = lax.dot_general(
                q4t, k5, (((2,), (2,)), ((0,), (0,))),
                preferred_element_type=jnp.float32,
            )
            e = jnp.exp(s)
            r = 1.0 / jnp.sum(e, axis=-1, keepdims=True)
            ctx = lax.dot_general(
                e.astype(jnp.bfloat16), v5, (((2,), (1,)), ((0,), (0,))),
                preferred_element_type=jnp.float32,
            )
            ctx2 = (
                (ctx * r).reshape(B_PER, HQ_PER, SQ, DH)
                .transpose(0, 2, 1, 3)
                .reshape(TOK, HD_PER).astype(jnp.bfloat16)
            )
            return lax.dot_general(
                ctx2, woT, (((1,), (1,)), ((), ())),
                preferred_element_type=jnp.float32,
            )

        HALF = D_MODEL // 2

        def mk(src_ref, src_slot, dst_ref, slot, half, send, recv, dst_dev):
            rows = pl.ds(half * HALF, HALF)
            src = (src_ref.at[rows] if src_slot is None
                   else src_ref.at[src_slot, rows])
            return pltpu.make_async_remote_copy(
                src_ref=src,
                dst_ref=dst_ref.at[slot, rows],
                send_sem=send.at[slot, half],
                recv_sem=recv.at[slot, half],
                device_id=(dst_dev,),
                device_id_type=pl.DeviceIdType.MESH,
            )

        cw, ccw = [], []
        for half in (0, 1):
            cw.append(mk(w_ref, None, cw_ref, 0, half, cw_send, cw_recv, right))
            ccw.append(mk(w_ref, None, ccw_ref, 0, half, ccw_send, ccw_recv, left))
            cw[-1].start()
            ccw[-1].start()

        acc = compute(0, w_ref[...])

        for r in range(1, N_CW + 1):
            has_ccw = r <= N_CCW
            for half in (0, 1):
                cw[2 * (r - 1) + half].wait_recv()
                if r < N_CW:
                    nxt = mk(cw_ref, r - 1, cw_ref, r, half, cw_send, cw_recv, right)
                    nxt.start()
                    cw.append(nxt)
                if has_ccw:
                    ccw[2 * (r - 1) + half].wait_recv()
                    if r < N_CCW:
                        nxt = mk(ccw_ref, r - 1, ccw_ref, r, half,
                                 ccw_send, ccw_recv, left)
                        nxt.start()
                        ccw.append(nxt)
            acc = acc + compute(2 * r - 1 if r < N_CW else 15, cw_ref[r - 1])
            if has_ccw:
                acc = acc + compute(2 * r, ccw_ref[r - 1])

        for rdma in cw + ccw:
            rdma.wait_send()

        out_ref[...] = acc.reshape(B_PER, SQ, D_MODEL)

    return pl.pallas_call(
        body,
        out_shape=jax.ShapeDtypeStruct((B_PER, SQ, D_MODEL), jnp.float32),
        in_specs=[
            pl.BlockSpec(memory_space=pltpu.VMEM),
            pl.BlockSpec(memory_space=pltpu.VMEM),
            pl.BlockSpec(memory_space=pltpu.VMEM),
            pl.BlockSpec(memory_space=pltpu.VMEM),
        ],
        out_specs=pl.BlockSpec(memory_space=pltpu.VMEM),
        scratch_shapes=[
            pltpu.VMEM((N_CW, D_MODEL, 2 * HD_PER), jnp.bfloat16),
            pltpu.VMEM((N_CCW, D_MODEL, 2 * HD_PER), jnp.bfloat16),
            pltpu.SemaphoreType.DMA((N_CW, 2)),
            pltpu.SemaphoreType.DMA((N_CW, 2)),
            pltpu.SemaphoreType.DMA((N_CCW, 2)),
            pltpu.SemaphoreType.DMA((N_CCW, 2)),
        ],
        compiler_params=pltpu.CompilerParams(collective_id=0),
    )(x, w_cat, k_arr, v_arr)
